# baseline (speedup 1.0000x reference)
import functools

import jax
import jax.numpy as jnp
from jax import lax
from jax.experimental import pallas as pl
from jax.experimental.pallas import tpu as pltpu
from jax.experimental.pallas import tpu_sc as plsc

N = 10000
E = 320000
H = 128
S = 256
MAXN_C = 16

NC = 2
NS = 16
CHUNK = 50
NCHUNK = 200
ZCH = 640
ZTAIL = N - (NS - 1) * ZCH
N_PAD = NS * ZCH

_MESH = plsc.VectorSubcoreMesh(
    core_axis_name="c", subcore_axis_name="s", num_cores=NC, num_subcores=NS)


@functools.partial(
    pl.kernel,
    out_type=[jax.ShapeDtypeStruct((N_PAD,), jnp.float32),
              jax.ShapeDtypeStruct((N_PAD,), jnp.float32)],
    mesh=_MESH,
    scratch_types=[
        pltpu.VMEM((NCHUNK, CHUNK), jnp.int32),
        pltpu.VMEM((CHUNK,), jnp.float32),
        pltpu.VMEM((ZCH,), jnp.float32),
        pltpu.VMEM_SHARED((N_PAD,), jnp.float32),
        pltpu.SemaphoreType.DMA,
    ],
)
def _deg_kernel(dst_hbm, out0_hbm, out1_hbm, dst_v, ones_v, z_v, accum, dsem):
    c = lax.axis_index("c")
    s = lax.axis_index("s")
    w = s * NC + c
    for j in range(-(-CHUNK // 16)):
        ones_v[pl.ds(min(j * 16, CHUNK - 16), 16)] = jnp.ones((16,), jnp.float32)
    for j in range(ZCH // 16):
        z_v[pl.ds(j * 16, 16)] = jnp.zeros((16,), jnp.float32)

    pltpu.sync_copy(z_v, accum.at[pl.ds(s * ZCH, ZCH)])
    plsc.subcore_barrier()
    pltpu.sync_copy(dst_hbm.at[w], dst_v)

    DEGW = 16

    def body(i, _):
        pltpu.async_copy(ones_v, accum.at[dst_v.at[i]], dsem, add=True)

        @pl.when(i >= DEGW)
        def _():
            pltpu.make_async_copy(ones_v, accum.at[dst_v.at[i - DEGW]],
                                  dsem).wait()
        return ()

    lax.fori_loop(0, NCHUNK, body, ())
    for i in range(NCHUNK - DEGW, NCHUNK):
        pltpu.make_async_copy(ones_v, accum.at[dst_v.at[i]], dsem).wait()
    plsc.subcore_barrier()

    for cc, out_hbm in ((0, out0_hbm), (1, out1_hbm)):
        @pl.when(c == cc)
        def _(out_hbm=out_hbm):
            pltpu.sync_copy(accum.at[pl.ds(s * ZCH, ZCH)],
                            out_hbm.at[pl.ds(s * ZCH, ZCH)])


NB = 4
NFULL = (NCHUNK - 2) // NB


@functools.partial(
    pl.kernel,
    out_type=jax.ShapeDtypeStruct((NC, N, H), jnp.float32),
    mesh=_MESH,
    scratch_types=[
        pltpu.VMEM((NCHUNK, CHUNK), jnp.int32),
        pltpu.VMEM((NCHUNK, CHUNK), jnp.int32),
        [pltpu.VMEM((CHUNK, H), jnp.float32)] * NB,
        pltpu.VMEM_SHARED((N, H), jnp.float32),
        [pltpu.SemaphoreType.DMA] * NB,
        [pltpu.SemaphoreType.DMA] * NB,
    ],
    compiler_params=pltpu.CompilerParams(use_tc_tiling_on_sc=False),
)
def _agg_kernel(x_hbm, src_hbm, dst_hbm, out_hbm, sidx, didx, bufs, accum,
                ssems, gsems):
    c = lax.axis_index("c")
    s = lax.axis_index("s")
    w = s * NC + c

    @pl.when(s < NS - 1)
    def _():
        pltpu.sync_copy(x_hbm.at[pl.ds(s * ZCH, ZCH)],
                        accum.at[pl.ds(s * ZCH, ZCH)])

    @pl.when(s == NS - 1)
    def _():
        pltpu.sync_copy(x_hbm.at[pl.ds((NS - 1) * ZCH, ZTAIL)],
                        accum.at[pl.ds((NS - 1) * ZCH, ZTAIL)])

    plsc.subcore_barrier()
    pltpu.sync_copy(src_hbm.at[w], sidx)
    pltpu.sync_copy(dst_hbm.at[w], didx)

    def _g(i, b):
        pltpu.async_copy(x_hbm.at[sidx.at[i]], bufs[b], gsems[b])

    def _gwait(i, b):
        pltpu.make_async_copy(x_hbm.at[sidx.at[i]], bufs[b], gsems[b]).wait()

    def _s(i, b):
        pltpu.async_copy(bufs[b], accum.at[didx.at[i]], ssems[b], add=True)

    def _swait(i, b):
        pltpu.make_async_copy(bufs[b], accum.at[didx.at[i]], ssems[b]).wait()

    for b in range(NB):
        _g(b, b)
    _gwait(0, 0)
    _s(0, 0)

    def body(k, _):
        for b in range(NB):
            i = k * NB + b
            _swait(i - NB, b)
            _g(i, b)
            j = i - (NB - 1)
            _gwait(j, (b + 1) % NB)
            _s(j, (b + 1) % NB)
        return ()

    lax.fori_loop(1, NFULL, body, ())
    last = NFULL * NB - 1
    for j in range(last - (NB - 2), last + 1):
        _gwait(j, j % NB)
        _s(j, j % NB)
    for t in range(NFULL * NB, NCHUNK):
        b = t % NB
        _swait(t - NB, b)
        pltpu.sync_copy(x_hbm.at[sidx.at[t]], bufs[b])
        _s(t, b)
    for j in range(NCHUNK - NB, NCHUNK):
        _swait(j, j % NB)
    plsc.subcore_barrier()

    @pl.when(s < NS - 1)
    def _():
        pltpu.sync_copy(accum.at[pl.ds(s * ZCH, ZCH)],
                        out_hbm.at[c, pl.ds(s * ZCH, ZCH)])

    @pl.when(s == NS - 1)
    def _():
        pltpu.sync_copy(accum.at[pl.ds((NS - 1) * ZCH, ZTAIL)],
                        out_hbm.at[c, pl.ds((NS - 1) * ZCH, ZTAIL)])


BLK = 1000


def _l0_body(d0_ref, d1_ref, w1_ref, b1_ref, w2_ref, b2_ref, out_ref):
    d = d0_ref[...] + d1_ref[...]
    h = jax.nn.gelu((1.0 + d) * w1_ref[...] + b1_ref[...])
    out_ref[...] = jax.nn.gelu(
        jnp.dot(h, w2_ref[...], preferred_element_type=jnp.float32)
        + b2_ref[...])


def _mlp_body(x_ref, a_ref, w1_ref, b1_ref, w2_ref, b2_ref, out_ref):
    hin = a_ref[0] + a_ref[1] - x_ref[...]
    h = jax.nn.gelu(
        jnp.dot(hin, w1_ref[...], preferred_element_type=jnp.float32)
        + b1_ref[...])
    out_ref[...] = jax.nn.gelu(
        jnp.dot(h, w2_ref[...], preferred_element_type=jnp.float32)
        + b2_ref[...])


def _fcpool_body(x_ref, a_ref, w1_ref, b1_ref, w2_ref, b2_ref,
                 seg_ref, fc0w_ref, fc0b_ref, fc1w_ref, fc1b_ref,
                 pooled_ref, res_ref):
    i = pl.program_id(0)
    hin = a_ref[0] + a_ref[1] - x_ref[...]
    h = jax.nn.gelu(
        jnp.dot(hin, w1_ref[...], preferred_element_type=jnp.float32)
        + b1_ref[...])
    x2 = jax.nn.gelu(
        jnp.dot(h, w2_ref[...], preferred_element_type=jnp.float32)
        + b2_ref[...])
    x3 = jax.nn.gelu(
        jnp.dot(x2, fc0w_ref[...], preferred_element_type=jnp.float32)
        + fc0b_ref[...])
    onehot = (seg_ref[...] ==
              lax.broadcasted_iota(jnp.int32, (1, S), 1)).astype(jnp.float32)
    contrib = lax.dot_general(onehot, x3, (((0,), (0,)), ((), ())),
                              preferred_element_type=jnp.float32)

    @pl.when(i == 0)
    def _():
        pooled_ref[...] = contrib

    @pl.when(i > 0)
    def _():
        pooled_ref[...] += contrib

    @pl.when(i == pl.num_programs(0) - 1)
    def _():
        res_ref[...] = (
            jnp.dot(pooled_ref[...], fc1w_ref[...],
                    preferred_element_type=jnp.float32) + fc1b_ref[...])


_W_SPEC = lambda shp: pl.BlockSpec(shp, lambda i: (0, 0))
_ARB = pltpu.CompilerParams(dimension_semantics=("arbitrary",))


def _l0_call(d0, d1, w1, b1, w2, b2):
    return pl.pallas_call(
        _l0_body,
        grid=(N // BLK,),
        in_specs=[
            pl.BlockSpec((BLK, 1), lambda i: (i, 0)),
            pl.BlockSpec((BLK, 1), lambda i: (i, 0)),
            _W_SPEC((1, H)), _W_SPEC((1, H)), _W_SPEC((H, H)), _W_SPEC((1, H)),
        ],
        out_specs=pl.BlockSpec((BLK, H), lambda i: (i, 0)),
        out_shape=jax.ShapeDtypeStruct((N, H), jnp.float32),
        compiler_params=_ARB,
    )(d0, d1, w1, b1, w2, b2)


def _mlp_call(x, a, w1, b1, w2, b2):
    return pl.pallas_call(
        _mlp_body,
        grid=(N // BLK,),
        in_specs=[
            pl.BlockSpec((BLK, H), lambda i: (i, 0)),
            pl.BlockSpec((NC, BLK, H), lambda i: (0, i, 0)),
            _W_SPEC((H, H)), _W_SPEC((1, H)), _W_SPEC((H, H)), _W_SPEC((1, H)),
        ],
        out_specs=pl.BlockSpec((BLK, H), lambda i: (i, 0)),
        out_shape=jax.ShapeDtypeStruct((N, H), jnp.float32),
        compiler_params=_ARB,
    )(x, a, w1, b1, w2, b2)


def _fcpool_call(x, a, w1, b1, w2, b2, seg, fc0w, fc0b, fc1w, fc1b):
    return pl.pallas_call(
        _fcpool_body,
        grid=(N // BLK,),
        in_specs=[
            pl.BlockSpec((BLK, H), lambda i: (i, 0)),
            pl.BlockSpec((NC, BLK, H), lambda i: (0, i, 0)),
            _W_SPEC((H, H)), _W_SPEC((1, H)), _W_SPEC((H, H)), _W_SPEC((1, H)),
            pl.BlockSpec((BLK, 1), lambda i: (i, 0)),
            _W_SPEC((H, H)), _W_SPEC((1, H)), _W_SPEC((H, 1)), _W_SPEC((1, 1)),
        ],
        out_specs=[
            pl.BlockSpec((S, H), lambda i: (0, 0)),
            pl.BlockSpec((S, 1), lambda i: (0, 0)),
        ],
        out_shape=[
            jax.ShapeDtypeStruct((S, H), jnp.float32),
            jax.ShapeDtypeStruct((S, 1), jnp.float32),
        ],
        compiler_params=_ARB,
    )(x, a, w1, b1, w2, b2, seg, fc0w, fc0b, fc1w, fc1b)


def kernel(edge_index, node2subgraph, subgraph_ids, max_nodes,
           gin0_W1, gin0_b1, gin0_W2, gin0_b2,
           gin1_W1, gin1_b1, gin1_W2, gin1_b2,
           gin2_W1, gin2_b1, gin2_W2, gin2_b2,
           fc0_W, fc0_b, fc1_W, fc1_b):
    src2 = edge_index[0].reshape(NC * NS, NCHUNK, CHUNK).astype(jnp.int32)
    dst2 = edge_index[1].reshape(NC * NS, NCHUNK, CHUNK).astype(jnp.int32)

    d0, d1 = _deg_kernel(dst2)
    x0 = _l0_call(d0[:N].reshape(N, 1), d1[:N].reshape(N, 1), gin0_W1,
                  gin0_b1.reshape(1, H), gin0_W2, gin0_b2.reshape(1, H))
    a1 = _agg_kernel(x0, src2, dst2)
    x1 = _mlp_call(x0, a1, gin1_W1, gin1_b1.reshape(1, H),
                   gin1_W2, gin1_b2.reshape(1, H))
    a2 = _agg_kernel(x1, src2, dst2)
    pooled, res = _fcpool_call(
        x1, a2, gin2_W1, gin2_b1.reshape(1, H),
        gin2_W2, gin2_b2.reshape(1, H),
        node2subgraph.reshape(N, 1).astype(jnp.int32),
        fc0_W, fc0_b.reshape(1, H), fc1_W, fc1_b.reshape(1, 1))
    return pooled, res.reshape(S // MAXN_C, MAXN_C)


# speedup vs baseline: 1.0589x; 1.0589x over previous
import functools

import jax
import jax.numpy as jnp
from jax import lax
from jax.experimental import pallas as pl
from jax.experimental.pallas import tpu as pltpu
from jax.experimental.pallas import tpu_sc as plsc

N = 10000
E = 320000
H = 128
S = 256
MAXN_C = 16

NC = 2
NS = 16
CHUNK = 80
NCHUNK = 125
ZCH = 640
ZTAIL = N - (NS - 1) * ZCH
N_PAD = NS * ZCH

_MESH = plsc.VectorSubcoreMesh(
    core_axis_name="c", subcore_axis_name="s", num_cores=NC, num_subcores=NS)


@functools.partial(
    pl.kernel,
    out_type=[jax.ShapeDtypeStruct((N_PAD,), jnp.float32),
              jax.ShapeDtypeStruct((N_PAD,), jnp.float32)],
    mesh=_MESH,
    scratch_types=[
        pltpu.VMEM((NCHUNK, CHUNK), jnp.int32),
        pltpu.VMEM((CHUNK,), jnp.float32),
        pltpu.VMEM((ZCH,), jnp.float32),
        pltpu.VMEM_SHARED((N_PAD,), jnp.float32),
        pltpu.SemaphoreType.DMA,
    ],
)
def _deg_kernel(dst_hbm, out0_hbm, out1_hbm, dst_v, ones_v, z_v, accum, dsem):
    c = lax.axis_index("c")
    s = lax.axis_index("s")
    w = s * NC + c
    for j in range(-(-CHUNK // 16)):
        ones_v[pl.ds(min(j * 16, CHUNK - 16), 16)] = jnp.ones((16,), jnp.float32)
    for j in range(ZCH // 16):
        z_v[pl.ds(j * 16, 16)] = jnp.zeros((16,), jnp.float32)

    pltpu.sync_copy(z_v, accum.at[pl.ds(s * ZCH, ZCH)])
    plsc.subcore_barrier()
    pltpu.sync_copy(dst_hbm.at[w], dst_v)

    DEGW = 16

    def body(i, _):
        pltpu.async_copy(ones_v, accum.at[dst_v.at[i]], dsem, add=True)

        @pl.when(i >= DEGW)
        def _():
            pltpu.make_async_copy(ones_v, accum.at[dst_v.at[i - DEGW]],
                                  dsem).wait()
        return ()

    lax.fori_loop(0, NCHUNK, body, ())
    for i in range(NCHUNK - DEGW, NCHUNK):
        pltpu.make_async_copy(ones_v, accum.at[dst_v.at[i]], dsem).wait()
    plsc.subcore_barrier()

    for cc, out_hbm in ((0, out0_hbm), (1, out1_hbm)):
        @pl.when(c == cc)
        def _(out_hbm=out_hbm):
            pltpu.sync_copy(accum.at[pl.ds(s * ZCH, ZCH)],
                            out_hbm.at[pl.ds(s * ZCH, ZCH)])


NB = 3
NFULL = (NCHUNK - 2) // NB


@functools.partial(
    pl.kernel,
    out_type=jax.ShapeDtypeStruct((NC, N, H), jnp.float32),
    mesh=_MESH,
    scratch_types=[
        pltpu.VMEM((NCHUNK, CHUNK), jnp.int32),
        pltpu.VMEM((NCHUNK, CHUNK), jnp.int32),
        [pltpu.VMEM((CHUNK, H), jnp.float32)] * NB,
        pltpu.VMEM_SHARED((N, H), jnp.float32),
        [pltpu.SemaphoreType.DMA] * NB,
        [pltpu.SemaphoreType.DMA] * NB,
    ],
    compiler_params=pltpu.CompilerParams(use_tc_tiling_on_sc=False),
)
def _agg_kernel(x_hbm, src_hbm, dst_hbm, out_hbm, sidx, didx, bufs, accum,
                ssems, gsems):
    c = lax.axis_index("c")
    s = lax.axis_index("s")
    w = s * NC + c

    @pl.when(s < NS - 1)
    def _():
        pltpu.sync_copy(x_hbm.at[pl.ds(s * ZCH, ZCH)],
                        accum.at[pl.ds(s * ZCH, ZCH)])

    @pl.when(s == NS - 1)
    def _():
        pltpu.sync_copy(x_hbm.at[pl.ds((NS - 1) * ZCH, ZTAIL)],
                        accum.at[pl.ds((NS - 1) * ZCH, ZTAIL)])

    plsc.subcore_barrier()
    pltpu.sync_copy(src_hbm.at[w], sidx)
    pltpu.sync_copy(dst_hbm.at[w], didx)

    def _g(i, b):
        pltpu.async_copy(x_hbm.at[sidx.at[i]], bufs[b], gsems[b])

    def _gwait(i, b):
        pltpu.make_async_copy(x_hbm.at[sidx.at[i]], bufs[b], gsems[b]).wait()

    def _s(i, b):
        pltpu.async_copy(bufs[b], accum.at[didx.at[i]], ssems[b], add=True)

    def _swait(i, b):
        pltpu.make_async_copy(bufs[b], accum.at[didx.at[i]], ssems[b]).wait()

    for b in range(NB):
        _g(b, b)
    _gwait(0, 0)
    _s(0, 0)

    def body(k, _):
        for b in range(NB):
            i = k * NB + b
            _swait(i - NB, b)
            _g(i, b)
            j = i - (NB - 1)
            _gwait(j, (b + 1) % NB)
            _s(j, (b + 1) % NB)
        return ()

    lax.fori_loop(1, NFULL, body, ())
    last = NFULL * NB - 1
    for j in range(last - (NB - 2), last + 1):
        _gwait(j, j % NB)
        _s(j, j % NB)
    for t in range(NFULL * NB, NCHUNK):
        b = t % NB
        _swait(t - NB, b)
        pltpu.sync_copy(x_hbm.at[sidx.at[t]], bufs[b])
        _s(t, b)
    for j in range(NCHUNK - NB, NCHUNK):
        _swait(j, j % NB)
    plsc.subcore_barrier()

    @pl.when(s < NS - 1)
    def _():
        pltpu.sync_copy(accum.at[pl.ds(s * ZCH, ZCH)],
                        out_hbm.at[c, pl.ds(s * ZCH, ZCH)])

    @pl.when(s == NS - 1)
    def _():
        pltpu.sync_copy(accum.at[pl.ds((NS - 1) * ZCH, ZTAIL)],
                        out_hbm.at[c, pl.ds((NS - 1) * ZCH, ZTAIL)])


BLK = 1000


def _l0_body(d0_ref, d1_ref, w1_ref, b1_ref, w2_ref, b2_ref, out_ref):
    d = d0_ref[...] + d1_ref[...]
    h = jax.nn.gelu((1.0 + d) * w1_ref[...] + b1_ref[...])
    out_ref[...] = jax.nn.gelu(
        jnp.dot(h, w2_ref[...], preferred_element_type=jnp.float32)
        + b2_ref[...])


def _mlp_body(x_ref, a_ref, w1_ref, b1_ref, w2_ref, b2_ref, out_ref):
    hin = a_ref[0] + a_ref[1] - x_ref[...]
    h = jax.nn.gelu(
        jnp.dot(hin, w1_ref[...], preferred_element_type=jnp.float32)
        + b1_ref[...])
    out_ref[...] = jax.nn.gelu(
        jnp.dot(h, w2_ref[...], preferred_element_type=jnp.float32)
        + b2_ref[...])


def _fcpool_body(x_ref, a_ref, w1_ref, b1_ref, w2_ref, b2_ref,
                 seg_ref, fc0w_ref, fc0b_ref, fc1w_ref, fc1b_ref,
                 pooled_ref, res_ref):
    i = pl.program_id(0)
    hin = a_ref[0] + a_ref[1] - x_ref[...]
    h = jax.nn.gelu(
        jnp.dot(hin, w1_ref[...], preferred_element_type=jnp.float32)
        + b1_ref[...])
    x2 = jax.nn.gelu(
        jnp.dot(h, w2_ref[...], preferred_element_type=jnp.float32)
        + b2_ref[...])
    x3 = jax.nn.gelu(
        jnp.dot(x2, fc0w_ref[...], preferred_element_type=jnp.float32)
        + fc0b_ref[...])
    onehot = (seg_ref[...] ==
              lax.broadcasted_iota(jnp.int32, (1, S), 1)).astype(jnp.float32)
    contrib = lax.dot_general(onehot, x3, (((0,), (0,)), ((), ())),
                              preferred_element_type=jnp.float32)

    @pl.when(i == 0)
    def _():
        pooled_ref[...] = contrib

    @pl.when(i > 0)
    def _():
        pooled_ref[...] += contrib

    @pl.when(i == pl.num_programs(0) - 1)
    def _():
        res_ref[...] = (
            jnp.dot(pooled_ref[...], fc1w_ref[...],
                    preferred_element_type=jnp.float32) + fc1b_ref[...])


_W_SPEC = lambda shp: pl.BlockSpec(shp, lambda i: (0, 0))
_ARB = pltpu.CompilerParams(dimension_semantics=("arbitrary",))


def _l0_call(d0, d1, w1, b1, w2, b2):
    return pl.pallas_call(
        _l0_body,
        grid=(N // BLK,),
        in_specs=[
            pl.BlockSpec((BLK, 1), lambda i: (i, 0)),
            pl.BlockSpec((BLK, 1), lambda i: (i, 0)),
            _W_SPEC((1, H)), _W_SPEC((1, H)), _W_SPEC((H, H)), _W_SPEC((1, H)),
        ],
        out_specs=pl.BlockSpec((BLK, H), lambda i: (i, 0)),
        out_shape=jax.ShapeDtypeStruct((N, H), jnp.float32),
        compiler_params=_ARB,
    )(d0, d1, w1, b1, w2, b2)


def _mlp_call(x, a, w1, b1, w2, b2):
    return pl.pallas_call(
        _mlp_body,
        grid=(N // BLK,),
        in_specs=[
            pl.BlockSpec((BLK, H), lambda i: (i, 0)),
            pl.BlockSpec((NC, BLK, H), lambda i: (0, i, 0)),
            _W_SPEC((H, H)), _W_SPEC((1, H)), _W_SPEC((H, H)), _W_SPEC((1, H)),
        ],
        out_specs=pl.BlockSpec((BLK, H), lambda i: (i, 0)),
        out_shape=jax.ShapeDtypeStruct((N, H), jnp.float32),
        compiler_params=_ARB,
    )(x, a, w1, b1, w2, b2)


def _fcpool_call(x, a, w1, b1, w2, b2, seg, fc0w, fc0b, fc1w, fc1b):
    return pl.pallas_call(
        _fcpool_body,
        grid=(N // BLK,),
        in_specs=[
            pl.BlockSpec((BLK, H), lambda i: (i, 0)),
            pl.BlockSpec((NC, BLK, H), lambda i: (0, i, 0)),
            _W_SPEC((H, H)), _W_SPEC((1, H)), _W_SPEC((H, H)), _W_SPEC((1, H)),
            pl.BlockSpec((BLK, 1), lambda i: (i, 0)),
            _W_SPEC((H, H)), _W_SPEC((1, H)), _W_SPEC((H, 1)), _W_SPEC((1, 1)),
        ],
        out_specs=[
            pl.BlockSpec((S, H), lambda i: (0, 0)),
            pl.BlockSpec((S, 1), lambda i: (0, 0)),
        ],
        out_shape=[
            jax.ShapeDtypeStruct((S, H), jnp.float32),
            jax.ShapeDtypeStruct((S, 1), jnp.float32),
        ],
        compiler_params=_ARB,
    )(x, a, w1, b1, w2, b2, seg, fc0w, fc0b, fc1w, fc1b)


def kernel(edge_index, node2subgraph, subgraph_ids, max_nodes,
           gin0_W1, gin0_b1, gin0_W2, gin0_b2,
           gin1_W1, gin1_b1, gin1_W2, gin1_b2,
           gin2_W1, gin2_b1, gin2_W2, gin2_b2,
           fc0_W, fc0_b, fc1_W, fc1_b):
    src2 = edge_index[0].reshape(NC * NS, NCHUNK, CHUNK).astype(jnp.int32)
    dst2 = edge_index[1].reshape(NC * NS, NCHUNK, CHUNK).astype(jnp.int32)

    d0, d1 = _deg_kernel(dst2)
    x0 = _l0_call(d0[:N].reshape(N, 1), d1[:N].reshape(N, 1), gin0_W1,
                  gin0_b1.reshape(1, H), gin0_W2, gin0_b2.reshape(1, H))
    a1 = _agg_kernel(x0, src2, dst2)
    x1 = _mlp_call(x0, a1, gin1_W1, gin1_b1.reshape(1, H),
                   gin1_W2, gin1_b2.reshape(1, H))
    a2 = _agg_kernel(x1, src2, dst2)
    pooled, res = _fcpool_call(
        x1, a2, gin2_W1, gin2_b1.reshape(1, H),
        gin2_W2, gin2_b2.reshape(1, H),
        node2subgraph.reshape(N, 1).astype(jnp.int32),
        fc0_W, fc0_b.reshape(1, H), fc1_W, fc1_b.reshape(1, 1))
    return pooled, res.reshape(S // MAXN_C, MAXN_C)


# speedup vs baseline: 1.0985x; 1.0374x over previous
import functools

import jax
import jax.numpy as jnp
from jax import lax
from jax.experimental import pallas as pl
from jax.experimental.pallas import tpu as pltpu
from jax.experimental.pallas import tpu_sc as plsc

N = 10000
E = 320000
H = 128
S = 256
MAXN_C = 16

NC = 2
NS = 16
CHUNK = 80
NCHUNK = 125
ZCH = 640
ZTAIL = N - (NS - 1) * ZCH
N_PAD = NS * ZCH

_MESH = plsc.VectorSubcoreMesh(
    core_axis_name="c", subcore_axis_name="s", num_cores=NC, num_subcores=NS)


@functools.partial(
    pl.kernel,
    out_type=[jax.ShapeDtypeStruct((N_PAD,), jnp.float32),
              jax.ShapeDtypeStruct((N_PAD,), jnp.float32)],
    mesh=_MESH,
    scratch_types=[
        pltpu.VMEM((NCHUNK, CHUNK), jnp.int32),
        pltpu.VMEM((CHUNK,), jnp.float32),
        pltpu.VMEM((ZCH,), jnp.float32),
        pltpu.VMEM_SHARED((N_PAD,), jnp.float32),
        pltpu.SemaphoreType.DMA,
    ],
)
def _deg_kernel(dst_hbm, out0_hbm, out1_hbm, dst_v, ones_v, z_v, accum, dsem):
    c = lax.axis_index("c")
    s = lax.axis_index("s")
    w = s * NC + c
    for j in range(-(-CHUNK // 16)):
        ones_v[pl.ds(min(j * 16, CHUNK - 16), 16)] = jnp.ones((16,), jnp.float32)
    for j in range(ZCH // 16):
        z_v[pl.ds(j * 16, 16)] = jnp.zeros((16,), jnp.float32)

    pltpu.sync_copy(z_v, accum.at[pl.ds(s * ZCH, ZCH)])
    plsc.subcore_barrier()
    pltpu.sync_copy(dst_hbm.at[w], dst_v)

    DEGW = 16

    def body(i, _):
        pltpu.async_copy(ones_v, accum.at[dst_v.at[i]], dsem, add=True)

        @pl.when(i >= DEGW)
        def _():
            pltpu.make_async_copy(ones_v, accum.at[dst_v.at[i - DEGW]],
                                  dsem).wait()
        return ()

    lax.fori_loop(0, NCHUNK, body, ())
    for i in range(NCHUNK - DEGW, NCHUNK):
        pltpu.make_async_copy(ones_v, accum.at[dst_v.at[i]], dsem).wait()
    plsc.subcore_barrier()

    for cc, out_hbm in ((0, out0_hbm), (1, out1_hbm)):
        @pl.when(c == cc)
        def _(out_hbm=out_hbm):
            pltpu.sync_copy(accum.at[pl.ds(s * ZCH, ZCH)],
                            out_hbm.at[pl.ds(s * ZCH, ZCH)])


NB = 3
NFULL = (NCHUNK - 2) // NB


@functools.partial(
    pl.kernel,
    out_type=jax.ShapeDtypeStruct((NC, N, H), jnp.float32),
    mesh=_MESH,
    scratch_types=[
        pltpu.VMEM((NCHUNK, CHUNK), jnp.int32),
        pltpu.VMEM((NCHUNK, CHUNK), jnp.int32),
        [pltpu.VMEM((CHUNK, H), jnp.float32)] * NB,
        pltpu.VMEM_SHARED((N, H), jnp.float32),
        [pltpu.SemaphoreType.DMA] * NB,
        [pltpu.SemaphoreType.DMA] * NB,
    ],
    compiler_params=pltpu.CompilerParams(use_tc_tiling_on_sc=False),
)
def _agg_kernel(x_hbm, src_hbm, dst_hbm, out_hbm, sidx, didx, bufs, accum,
                ssems, gsems):
    c = lax.axis_index("c")
    s = lax.axis_index("s")
    w = s * NC + c

    @pl.when(s < NS - 1)
    def _():
        pltpu.sync_copy(x_hbm.at[pl.ds(s * ZCH, ZCH)],
                        accum.at[pl.ds(s * ZCH, ZCH)])

    @pl.when(s == NS - 1)
    def _():
        pltpu.sync_copy(x_hbm.at[pl.ds((NS - 1) * ZCH, ZTAIL)],
                        accum.at[pl.ds((NS - 1) * ZCH, ZTAIL)])

    plsc.subcore_barrier()
    pltpu.sync_copy(src_hbm.at[w], sidx)
    pltpu.sync_copy(dst_hbm.at[w], didx)

    def _g(i, b):
        pltpu.async_copy(x_hbm.at[sidx.at[i]], bufs[b], gsems[b])

    def _gwait(i, b):
        pltpu.make_async_copy(x_hbm.at[sidx.at[i]], bufs[b], gsems[b]).wait()

    def _s(i, b):
        pltpu.async_copy(bufs[b], accum.at[didx.at[i]], ssems[b], add=True)

    def _swait(i, b):
        pltpu.make_async_copy(bufs[b], accum.at[didx.at[i]], ssems[b]).wait()

    for b in range(NB):
        _g(b, b)
    _gwait(0, 0)
    _s(0, 0)

    def body(k, _):
        for b in range(NB):
            i = k * NB + b
            _swait(i - NB, b)
            _g(i, b)
            j = i - (NB - 1)
            _gwait(j, (b + 1) % NB)
            _s(j, (b + 1) % NB)
        return ()

    lax.fori_loop(1, NFULL, body, ())
    last = NFULL * NB - 1
    for j in range(last - (NB - 2), last + 1):
        _gwait(j, j % NB)
        _s(j, j % NB)
    for t in range(NFULL * NB, NCHUNK):
        b = t % NB
        _swait(t - NB, b)
        pltpu.sync_copy(x_hbm.at[sidx.at[t]], bufs[b])
        _s(t, b)
    for j in range(NCHUNK - NB, NCHUNK):
        _swait(j, j % NB)
    plsc.subcore_barrier()

    @pl.when(s < NS - 1)
    def _():
        pltpu.sync_copy(accum.at[pl.ds(s * ZCH, ZCH)],
                        out_hbm.at[c, pl.ds(s * ZCH, ZCH)])

    @pl.when(s == NS - 1)
    def _():
        pltpu.sync_copy(accum.at[pl.ds((NS - 1) * ZCH, ZTAIL)],
                        out_hbm.at[c, pl.ds((NS - 1) * ZCH, ZTAIL)])


BLK = 2000


def _l0_body(d0_ref, d1_ref, w1_ref, b1_ref, w2_ref, b2_ref, out_ref):
    d = d0_ref[...] + d1_ref[...]
    h = jax.nn.gelu((1.0 + d) * w1_ref[...] + b1_ref[...])
    out_ref[...] = jax.nn.gelu(
        jnp.dot(h, w2_ref[...], preferred_element_type=jnp.float32)
        + b2_ref[...])


def _mlp_body(x_ref, a_ref, w1_ref, b1_ref, w2_ref, b2_ref, out_ref):
    hin = a_ref[0] + a_ref[1] - x_ref[...]
    h = jax.nn.gelu(
        jnp.dot(hin, w1_ref[...], preferred_element_type=jnp.float32)
        + b1_ref[...])
    out_ref[...] = jax.nn.gelu(
        jnp.dot(h, w2_ref[...], preferred_element_type=jnp.float32)
        + b2_ref[...])


def _fcpool_body(x_ref, a_ref, w1_ref, b1_ref, w2_ref, b2_ref,
                 seg_ref, fc0w_ref, fc0b_ref, fc1w_ref, fc1b_ref,
                 pooled_ref, res_ref):
    i = pl.program_id(0)
    hin = a_ref[0] + a_ref[1] - x_ref[...]
    h = jax.nn.gelu(
        jnp.dot(hin, w1_ref[...], preferred_element_type=jnp.float32)
        + b1_ref[...])
    x2 = jax.nn.gelu(
        jnp.dot(h, w2_ref[...], preferred_element_type=jnp.float32)
        + b2_ref[...])
    x3 = jax.nn.gelu(
        jnp.dot(x2, fc0w_ref[...], preferred_element_type=jnp.float32)
        + fc0b_ref[...])
    onehot = (seg_ref[...] ==
              lax.broadcasted_iota(jnp.int32, (1, S), 1)).astype(jnp.float32)
    contrib = lax.dot_general(onehot, x3, (((0,), (0,)), ((), ())),
                              preferred_element_type=jnp.float32)

    @pl.when(i == 0)
    def _():
        pooled_ref[...] = contrib

    @pl.when(i > 0)
    def _():
        pooled_ref[...] += contrib

    @pl.when(i == pl.num_programs(0) - 1)
    def _():
        res_ref[...] = (
            jnp.dot(pooled_ref[...], fc1w_ref[...],
                    preferred_element_type=jnp.float32) + fc1b_ref[...])


_W_SPEC = lambda shp: pl.BlockSpec(shp, lambda i: (0, 0))
_ARB = pltpu.CompilerParams(dimension_semantics=("arbitrary",))


def _l0_call(d0, d1, w1, b1, w2, b2):
    return pl.pallas_call(
        _l0_body,
        grid=(N // BLK,),
        in_specs=[
            pl.BlockSpec((BLK, 1), lambda i: (i, 0)),
            pl.BlockSpec((BLK, 1), lambda i: (i, 0)),
            _W_SPEC((1, H)), _W_SPEC((1, H)), _W_SPEC((H, H)), _W_SPEC((1, H)),
        ],
        out_specs=pl.BlockSpec((BLK, H), lambda i: (i, 0)),
        out_shape=jax.ShapeDtypeStruct((N, H), jnp.float32),
        compiler_params=_ARB,
    )(d0, d1, w1, b1, w2, b2)


def _mlp_call(x, a, w1, b1, w2, b2):
    return pl.pallas_call(
        _mlp_body,
        grid=(N // BLK,),
        in_specs=[
            pl.BlockSpec((BLK, H), lambda i: (i, 0)),
            pl.BlockSpec((NC, BLK, H), lambda i: (0, i, 0)),
            _W_SPEC((H, H)), _W_SPEC((1, H)), _W_SPEC((H, H)), _W_SPEC((1, H)),
        ],
        out_specs=pl.BlockSpec((BLK, H), lambda i: (i, 0)),
        out_shape=jax.ShapeDtypeStruct((N, H), jnp.float32),
        compiler_params=_ARB,
    )(x, a, w1, b1, w2, b2)


def _fcpool_call(x, a, w1, b1, w2, b2, seg, fc0w, fc0b, fc1w, fc1b):
    return pl.pallas_call(
        _fcpool_body,
        grid=(N // BLK,),
        in_specs=[
            pl.BlockSpec((BLK, H), lambda i: (i, 0)),
            pl.BlockSpec((NC, BLK, H), lambda i: (0, i, 0)),
            _W_SPEC((H, H)), _W_SPEC((1, H)), _W_SPEC((H, H)), _W_SPEC((1, H)),
            pl.BlockSpec((BLK, 1), lambda i: (i, 0)),
            _W_SPEC((H, H)), _W_SPEC((1, H)), _W_SPEC((H, 1)), _W_SPEC((1, 1)),
        ],
        out_specs=[
            pl.BlockSpec((S, H), lambda i: (0, 0)),
            pl.BlockSpec((S, 1), lambda i: (0, 0)),
        ],
        out_shape=[
            jax.ShapeDtypeStruct((S, H), jnp.float32),
            jax.ShapeDtypeStruct((S, 1), jnp.float32),
        ],
        compiler_params=_ARB,
    )(x, a, w1, b1, w2, b2, seg, fc0w, fc0b, fc1w, fc1b)


def kernel(edge_index, node2subgraph, subgraph_ids, max_nodes,
           gin0_W1, gin0_b1, gin0_W2, gin0_b2,
           gin1_W1, gin1_b1, gin1_W2, gin1_b2,
           gin2_W1, gin2_b1, gin2_W2, gin2_b2,
           fc0_W, fc0_b, fc1_W, fc1_b):
    src2 = edge_index[0].reshape(NC * NS, NCHUNK, CHUNK).astype(jnp.int32)
    dst2 = edge_index[1].reshape(NC * NS, NCHUNK, CHUNK).astype(jnp.int32)

    d0, d1 = _deg_kernel(dst2)
    x0 = _l0_call(d0.reshape(N_PAD, 1), d1.reshape(N_PAD, 1), gin0_W1,
                  gin0_b1.reshape(1, H), gin0_W2, gin0_b2.reshape(1, H))
    a1 = _agg_kernel(x0, src2, dst2)
    x1 = _mlp_call(x0, a1, gin1_W1, gin1_b1.reshape(1, H),
                   gin1_W2, gin1_b2.reshape(1, H))
    a2 = _agg_kernel(x1, src2, dst2)
    pooled, res = _fcpool_call(
        x1, a2, gin2_W1, gin2_b1.reshape(1, H),
        gin2_W2, gin2_b2.reshape(1, H),
        node2subgraph.reshape(N, 1).astype(jnp.int32),
        fc0_W, fc0_b.reshape(1, H), fc1_W, fc1_b.reshape(1, 1))
    return pooled, res.reshape(S // MAXN_C, MAXN_C)
